# baseline (speedup 1.0000x reference)
import jax
import jax.numpy as jnp
from jax import lax
from jax.experimental import pallas as pl
from jax.experimental.pallas import tpu as pltpu
from jax.experimental.pallas import tpu_sc as plsc

_I = 65536
_J = 64
_F = 16
_E = _I * _J
_NC = 2
_NS = 16
_NW = _NC * _NS
_IPW = _I // _NW
_C = 32
_NCHUNK = _IPW // _C
_CR = _C * _J


def _sc_pass_body(u_hbm, b_hbm, wt_hbm, out_hbm,
                  ub0, ub1, bb0, bb1, wtm, st, lscr, cscr,
                  semu0, semu1, semb0, semb1):
    wid = lax.axis_index("s") * _NC + lax.axis_index("c")
    wrow = wid * _IPW * _J

    ubufs = (ub0, ub1)
    bbufs = (bb0, bb1)
    semus = (semu0, semu1)
    sembs = (semb0, semb1)

    lane = lax.iota(jnp.int32, 16)
    jgl = [jg * 16 + lane for jg in range(4)]
    cols = [jnp.full((16,), f, jnp.int32) for f in range(_F)]
    zcol = jnp.zeros((16,), jnp.int32)
    zvec = jnp.zeros((_F,), jnp.float32)

    pltpu.sync_copy(wt_hbm, wtm)
    for f in range(_F):
        for jg in range(4):
            st[f, pl.ds(jg * 16, 16)] = zvec

    def _start(g, ph):
        base = wrow + g * _CR
        pltpu.async_copy(u_hbm.at[pl.ds(base, _CR)], ubufs[ph], semus[ph])
        pltpu.async_copy(b_hbm.at[pl.ds(base, _CR)], bbufs[ph], sembs[ph])

    def _wait(ph):
        pltpu.make_async_copy(u_hbm.at[pl.ds(0, _CR)], ubufs[ph], semus[ph]).wait()
        pltpu.make_async_copy(b_hbm.at[pl.ds(0, _CR)], bbufs[ph], sembs[ph]).wait()

    _start(0, 0)

    def _process_chunk(ub, bb):
        for jg in range(4):
            wtvs = [wtm[f, pl.ds(jg * 16, 16)] for f in range(_F)]

            def _dot(ci, _, jg=jg, wtvs=wtvs):
                rowv = jgl[jg] + ci * _J
                acc = plsc.load_gather(bb, [rowv, zcol])
                for f in range(_F):
                    g = plsc.load_gather(ub, [rowv, cols[f]])
                    acc = acc + g * wtvs[f]
                lscr[ci, pl.ds(jg * 16, 16)] = acc
                return 0
            lax.fori_loop(0, _C, _dot, 0, unroll=2)

        def _smax(ci, _):
            ls = [lscr[ci, pl.ds(jg * 16, 16)] for jg in range(4)]
            mv = jnp.maximum(jnp.maximum(ls[0], ls[1]),
                             jnp.maximum(ls[2], ls[3]))
            m = jnp.max(mv)
            es = [jnp.exp(l - m) for l in ls]
            z = (jnp.sum(es[0]) + jnp.sum(es[1])
                 + jnp.sum(es[2]) + jnp.sum(es[3]))
            rzv = jnp.full((16,), 1.0, jnp.float32) / jnp.full((16,), z, jnp.float32)
            for jg in range(4):
                cscr[ci, pl.ds(jg * 16, 16)] = es[jg] * rzv
            return 0
        lax.fori_loop(0, _C, _smax, 0, unroll=2)

        for jg in range(4):
            def _msg(ci, accs, jg=jg):
                rowv = jgl[jg] + ci * _J
                cv = cscr[ci, pl.ds(jg * 16, 16)]
                return tuple(
                    accs[f] + cv * plsc.load_gather(ub, [rowv, cols[f]])
                    for f in range(_F))
            accs0 = tuple(st[f, pl.ds(jg * 16, 16)] for f in range(_F))
            accs = lax.fori_loop(0, _C, _msg, accs0, unroll=2)
            for f in range(_F):
                st[f, pl.ds(jg * 16, 16)] = accs[f]

    def _chunk_loop(g2, _):
        for ph in range(2):
            g = g2 * 2 + ph

            @pl.when(g + 1 < _NCHUNK)
            def _():
                _start(g + 1, 1 - ph)

            _wait(ph)
            _process_chunk(ubufs[ph], bbufs[ph])
        return 0

    lax.fori_loop(0, _NCHUNK // 2, _chunk_loop, 0)

    pltpu.sync_copy(st, out_hbm.at[wid])


_sc_pass = pl.kernel(
    _sc_pass_body,
    out_type=jax.ShapeDtypeStruct((_NW, _F, _J), jnp.float32),
    mesh=plsc.VectorSubcoreMesh(core_axis_name="c", subcore_axis_name="s",
                                num_cores=_NC, num_subcores=_NS),
    scratch_types=[
        pltpu.VMEM((_CR, _F), jnp.float32),
        pltpu.VMEM((_CR, _F), jnp.float32),
        pltpu.VMEM((_CR, 1), jnp.float32),
        pltpu.VMEM((_CR, 1), jnp.float32),
        pltpu.VMEM((_F, _J), jnp.float32),
        pltpu.VMEM((_F, _J), jnp.float32),
        pltpu.VMEM((_C, _J), jnp.float32),
        pltpu.VMEM((_C, _J), jnp.float32),
        pltpu.SemaphoreType.DMA,
        pltpu.SemaphoreType.DMA,
        pltpu.SemaphoreType.DMA,
        pltpu.SemaphoreType.DMA,
    ],
    compiler_params=pltpu.CompilerParams(use_tc_tiling_on_sc=False,
                                         needs_layout_passes=False),
    name="dgl_routing_sc_pass",
)


def _combine_body(sp_ref, wt_ref, v_ref, wn_ref):
    s = jnp.sum(sp_ref[...], axis=0)
    sq = jnp.sum(s * s, axis=0, keepdims=True)
    v = sq / (1.0 + sq) * (s / jnp.sqrt(sq))
    v_ref[...] = v
    wn_ref[...] = wt_ref[...] + v


_combine = pl.pallas_call(
    _combine_body,
    out_shape=(jax.ShapeDtypeStruct((_F, _J), jnp.float32),
               jax.ShapeDtypeStruct((_F, _J), jnp.float32)),
    name="dgl_routing_squash",
)


def kernel(u_hat, b, routing_num):
    def body(_, carry):
        vt, wt = carry
        sp = _sc_pass(u_hat, b, wt)
        vt, wt = _combine(sp, wt)
        return vt, wt

    v0 = jnp.zeros((_F, _J), jnp.float32)
    w0 = jnp.zeros((_F, _J), jnp.float32)
    vt, _ = lax.fori_loop(0, routing_num, body, (v0, w0))
    return vt.T


# speedup vs baseline: 2.1444x; 2.1444x over previous
import jax
import jax.numpy as jnp
from jax import lax
from jax.experimental import pallas as pl
from jax.experimental.pallas import tpu as pltpu
from jax.experimental.pallas import tpu_sc as plsc

_I = 65536
_J = 64
_F = 16
_NC = 2
_NS = 16
_NW = _NC * _NS
_IPW = _I // _NW
_C = 32
_NCHUNK = _IPW // _C


def _sc_pass_body(u_hbm, b_hbm, wt_hbm, out_hbm,
                  ub0, ub1, bb0, bb1, wtm, st, lscr, cscr,
                  semu0, semu1, semb0, semb1):
    wid = lax.axis_index("s") * _NC + lax.axis_index("c")
    wbase = wid * _IPW

    ubufs = (ub0, ub1)
    bbufs = (bb0, bb1)
    semus = (semu0, semu1)
    sembs = (semb0, semb1)

    zvec = jnp.zeros((_F,), jnp.float32)

    pltpu.sync_copy(wt_hbm, wtm)
    for f in range(_F):
        for jg in range(4):
            st[f, pl.ds(jg * 16, 16)] = zvec

    def _start(g, ph):
        base = wbase + g * _C
        pltpu.async_copy(u_hbm.at[pl.ds(base, _C)], ubufs[ph], semus[ph])
        pltpu.async_copy(b_hbm.at[pl.ds(base, _C)], bbufs[ph], sembs[ph])

    def _wait(ph):
        pltpu.make_async_copy(u_hbm.at[pl.ds(0, _C)], ubufs[ph], semus[ph]).wait()
        pltpu.make_async_copy(b_hbm.at[pl.ds(0, _C)], bbufs[ph], sembs[ph]).wait()

    _start(0, 0)

    def _process_chunk(ub, bb):
        for jg in range(4):
            sl = pl.ds(jg * 16, 16)
            wtvs = [wtm[f, sl] for f in range(_F)]

            def _dot(ci, _, sl=sl, wtvs=wtvs):
                acc = bb[ci, sl]
                for f in range(_F):
                    acc = acc + ub[ci, f, sl] * wtvs[f]
                lscr[ci, sl] = acc
                return 0
            lax.fori_loop(0, _C, _dot, 0, unroll=2)

        def _smax(ci, _):
            ls = [lscr[ci, pl.ds(jg * 16, 16)] for jg in range(4)]
            mv = jnp.maximum(jnp.maximum(ls[0], ls[1]),
                             jnp.maximum(ls[2], ls[3]))
            m = jnp.max(mv)
            es = [jnp.exp(l - m) for l in ls]
            z = (jnp.sum(es[0]) + jnp.sum(es[1])
                 + jnp.sum(es[2]) + jnp.sum(es[3]))
            rzv = jnp.full((16,), 1.0, jnp.float32) / jnp.full((16,), z, jnp.float32)
            for jg in range(4):
                cscr[ci, pl.ds(jg * 16, 16)] = es[jg] * rzv
            return 0
        lax.fori_loop(0, _C, _smax, 0, unroll=2)

        for jg in range(4):
            sl = pl.ds(jg * 16, 16)

            def _msg(ci, accs, sl=sl):
                cv = cscr[ci, sl]
                return tuple(accs[f] + cv * ub[ci, f, sl] for f in range(_F))
            accs0 = tuple(st[f, sl] for f in range(_F))
            accs = lax.fori_loop(0, _C, _msg, accs0, unroll=2)
            for f in range(_F):
                st[f, sl] = accs[f]

    def _chunk_loop(g2, _):
        for ph in range(2):
            g = g2 * 2 + ph

            @pl.when(g + 1 < _NCHUNK)
            def _():
                _start(g + 1, 1 - ph)

            _wait(ph)
            _process_chunk(ubufs[ph], bbufs[ph])
        return 0

    lax.fori_loop(0, _NCHUNK // 2, _chunk_loop, 0)

    pltpu.sync_copy(st, out_hbm.at[wid])


_sc_pass = pl.kernel(
    _sc_pass_body,
    out_type=jax.ShapeDtypeStruct((_NW, _F, _J), jnp.float32),
    mesh=plsc.VectorSubcoreMesh(core_axis_name="c", subcore_axis_name="s",
                                num_cores=_NC, num_subcores=_NS),
    scratch_types=[
        pltpu.VMEM((_C, _F, _J), jnp.float32),
        pltpu.VMEM((_C, _F, _J), jnp.float32),
        pltpu.VMEM((_C, _J), jnp.float32),
        pltpu.VMEM((_C, _J), jnp.float32),
        pltpu.VMEM((_F, _J), jnp.float32),
        pltpu.VMEM((_F, _J), jnp.float32),
        pltpu.VMEM((_C, _J), jnp.float32),
        pltpu.VMEM((_C, _J), jnp.float32),
        pltpu.SemaphoreType.DMA,
        pltpu.SemaphoreType.DMA,
        pltpu.SemaphoreType.DMA,
        pltpu.SemaphoreType.DMA,
    ],
    compiler_params=pltpu.CompilerParams(use_tc_tiling_on_sc=False,
                                         needs_layout_passes=False),
    name="dgl_routing_sc_pass",
)


def _combine_body(sp_ref, wt_ref, v_ref, wn_ref):
    s = jnp.sum(sp_ref[...], axis=0)
    sq = jnp.sum(s * s, axis=0, keepdims=True)
    v = sq / (1.0 + sq) * (s / jnp.sqrt(sq))
    v_ref[...] = v
    wn_ref[...] = wt_ref[...] + v


_combine = pl.pallas_call(
    _combine_body,
    out_shape=(jax.ShapeDtypeStruct((_F, _J), jnp.float32),
               jax.ShapeDtypeStruct((_F, _J), jnp.float32)),
    name="dgl_routing_squash",
)


def kernel(u_hat, b, routing_num):
    ut = jnp.transpose(u_hat.reshape(_I, _J, _F), (0, 2, 1))
    b2 = b.reshape(_I, _J)

    def body(_, carry):
        vt, wt = carry
        sp = _sc_pass(ut, b2, wt)
        vt, wt = _combine(sp, wt)
        return vt, wt

    v0 = jnp.zeros((_F, _J), jnp.float32)
    w0 = jnp.zeros((_F, _J), jnp.float32)
    vt, _ = lax.fori_loop(0, routing_num, body, (v0, w0))
    return vt.T


# speedup vs baseline: 3.8236x; 1.7831x over previous
import jax
import jax.numpy as jnp
from jax import lax
from jax.experimental import pallas as pl
from jax.experimental.pallas import tpu as pltpu
from jax.experimental.pallas import tpu_sc as plsc

_I = 65536
_J = 64
_F = 16
_JF = _J * _F
_NC = 2
_NS = 16
_NW = _NC * _NS
_IPW = _I // _NW
_C = 32
_NCHUNK = _IPW // _C


def _sc_pass_body(u_hbm, b_hbm, wd_hbm, dcol_hbm, out_hbm,
                  ub0, ub1, bb0, bb1, wdm, std, lscr, cscr, dcolm,
                  semu0, semu1, semb0, semb1):
    wid = lax.axis_index("s") * _NC + lax.axis_index("c")
    wbase = wid * _IPW

    ubufs = (ub0, ub1)
    bbufs = (bb0, bb1)
    semus = (semu0, semu1)
    sembs = (semb0, semb1)

    zvec = jnp.zeros((_F,), jnp.float32)

    pltpu.sync_copy(wd_hbm, wdm)
    pltpu.sync_copy(dcol_hbm, dcolm)
    for k in range(_F):
        for jg in range(4):
            std[k, pl.ds(jg * 16, 16)] = zvec

    def _start(g, ph):
        base = wbase + g * _C
        pltpu.async_copy(u_hbm.at[pl.ds(base, _C)], ubufs[ph], semus[ph])
        pltpu.async_copy(b_hbm.at[pl.ds(base, _C)], bbufs[ph], sembs[ph])

    def _wait(ph):
        pltpu.make_async_copy(u_hbm.at[pl.ds(0, _C)], ubufs[ph], semus[ph]).wait()
        pltpu.make_async_copy(b_hbm.at[pl.ds(0, _C)], bbufs[ph], sembs[ph]).wait()

    _start(0, 0)

    def _process_chunk(ub, bb):
        for jg in range(4):
            sl = pl.ds(jg * 16, 16)
            colvs = [dcolm[jg, k, :] for k in range(_F)]
            wdvs = [wdm[k, sl] for k in range(_F)]

            def _dot(ci, _, sl=sl, colvs=colvs, wdvs=wdvs):
                row = jnp.full((16,), ci, jnp.int32)
                acc = bb[ci, sl]
                for k in range(_F):
                    acc = acc + plsc.load_gather(ub, [row, colvs[k]]) * wdvs[k]
                lscr[ci, sl] = acc
                return 0
            lax.fori_loop(0, _C, _dot, 0, unroll=2)

        def _smax(ci, _):
            ls = [lscr[ci, pl.ds(jg * 16, 16)] for jg in range(4)]
            mv = jnp.maximum(jnp.maximum(ls[0], ls[1]),
                             jnp.maximum(ls[2], ls[3]))
            m = jnp.max(mv)
            es = [jnp.exp(l - m) for l in ls]
            z = (jnp.sum(es[0]) + jnp.sum(es[1])
                 + jnp.sum(es[2]) + jnp.sum(es[3]))
            rzv = jnp.full((16,), 1.0, jnp.float32) / jnp.full((16,), z, jnp.float32)
            for jg in range(4):
                cscr[ci, pl.ds(jg * 16, 16)] = es[jg] * rzv
            return 0
        lax.fori_loop(0, _C, _smax, 0, unroll=2)

        for jg in range(4):
            sl = pl.ds(jg * 16, 16)
            colvs = [dcolm[jg, k, :] for k in range(_F)]

            def _msg(ci, accs, sl=sl, colvs=colvs):
                row = jnp.full((16,), ci, jnp.int32)
                cv = cscr[ci, sl]
                return tuple(
                    accs[k] + cv * plsc.load_gather(ub, [row, colvs[k]])
                    for k in range(_F))
            accs0 = tuple(std[k, sl] for k in range(_F))
            accs = lax.fori_loop(0, _C, _msg, accs0, unroll=2)
            for k in range(_F):
                std[k, sl] = accs[k]

    def _chunk_loop(g2, _):
        for ph in range(2):
            g = g2 * 2 + ph

            @pl.when(g + 1 < _NCHUNK)
            def _():
                _start(g + 1, 1 - ph)

            _wait(ph)
            _process_chunk(ubufs[ph], bbufs[ph])
        return 0

    lax.fori_loop(0, _NCHUNK // 2, _chunk_loop, 0)

    pltpu.sync_copy(std, out_hbm.at[wid])


_sc_pass = pl.kernel(
    _sc_pass_body,
    out_type=jax.ShapeDtypeStruct((_NW, _F, _J), jnp.float32),
    mesh=plsc.VectorSubcoreMesh(core_axis_name="c", subcore_axis_name="s",
                                num_cores=_NC, num_subcores=_NS),
    scratch_types=[
        pltpu.VMEM((_C, _JF), jnp.float32),
        pltpu.VMEM((_C, _JF), jnp.float32),
        pltpu.VMEM((_C, _J), jnp.float32),
        pltpu.VMEM((_C, _J), jnp.float32),
        pltpu.VMEM((_F, _J), jnp.float32),
        pltpu.VMEM((_F, _J), jnp.float32),
        pltpu.VMEM((_C, _J), jnp.float32),
        pltpu.VMEM((_C, _J), jnp.float32),
        pltpu.VMEM((4, _F, _F), jnp.int32),
        pltpu.SemaphoreType.DMA,
        pltpu.SemaphoreType.DMA,
        pltpu.SemaphoreType.DMA,
        pltpu.SemaphoreType.DMA,
    ],
    compiler_params=pltpu.CompilerParams(use_tc_tiling_on_sc=False,
                                         needs_layout_passes=False),
    name="dgl_routing_sc_pass",
)


def _combine_body(sp_ref, wd_ref, v_ref, wn_ref):
    s = jnp.sum(sp_ref[...], axis=0)
    sq = jnp.sum(s * s, axis=0, keepdims=True)
    v = sq / (1.0 + sq) * (s / jnp.sqrt(sq))
    v_ref[...] = v
    wn_ref[...] = wd_ref[...] + v


_combine = pl.pallas_call(
    _combine_body,
    out_shape=(jax.ShapeDtypeStruct((_F, _J), jnp.float32),
               jax.ShapeDtypeStruct((_F, _J), jnp.float32)),
    name="dgl_routing_squash",
)


def kernel(u_hat, b, routing_num):
    u2 = u_hat.reshape(_I, _JF)
    b2 = b.reshape(_I, _J)

    lane = jnp.arange(16, dtype=jnp.int32)
    kk = jnp.arange(16, dtype=jnp.int32)
    jgc = jnp.arange(4, dtype=jnp.int32)
    dcol = ((jgc[:, None, None] * 16 + lane[None, None, :]) * 16
            + (lane[None, None, :] + kk[None, :, None]) % 16)

    def body(_, carry):
        vd, wd = carry
        sp = _sc_pass(u2, b2, wd, dcol)
        vd, wd = _combine(sp, wd)
        return vd, wd

    v0 = jnp.zeros((_F, _J), jnp.float32)
    w0 = jnp.zeros((_F, _J), jnp.float32)
    vd, _ = lax.fori_loop(0, routing_num, body, (v0, w0))

    ff = jnp.arange(_F, dtype=jnp.int32)[:, None]
    cc = jnp.arange(_J, dtype=jnp.int32)[None, :]
    uidx = (ff - (cc % 16)) % 16
    v = jnp.take_along_axis(vd, uidx, axis=0)
    return v.T
